# baseline (speedup 1.0000x reference)
import jax
import jax.numpy as jnp
from jax.experimental import pallas as pl


def _sage_fused(adj_ref, gx_ref,
                wl0_ref, bl0_ref, wr0_ref,
                wl1_ref, bl1_ref, wr1_ref,
                wl2_ref, bl2_ref, wr2_ref,
                out_ref):
    adj = adj_ref[...]
    g0 = gx_ref[...]
    d = g0.shape[0] - 1

    first = jnp.dot(g0, adj, preferred_element_type=jnp.float32)
    dinv = 1.0 / jnp.maximum(first[d:, :], 1.0)

    def tail(gh, aggT, wlT_ref, blc_ref, wrT_ref):
        meanT = aggT * dinv
        lin_l = jnp.dot(wlT_ref[...], meanT, preferred_element_type=jnp.float32)
        lin_r = jnp.dot(wrT_ref[...], gh, preferred_element_type=jnp.float32)
        return lin_l + lin_r + blc_ref[...]

    g = jnp.maximum(tail(g0[:d, :].astype(jnp.float32), first[:d, :],
                         wl0_ref, bl0_ref, wr0_ref), 0.0)
    gb = g.astype(jnp.bfloat16)
    aggT = jnp.dot(gb, adj, preferred_element_type=jnp.float32)
    g = jnp.maximum(tail(g, aggT, wl1_ref, bl1_ref, wr1_ref), 0.0)
    gb = g.astype(jnp.bfloat16)
    aggT = jnp.dot(gb, adj, preferred_element_type=jnp.float32)
    out_ref[...] = tail(g, aggT, wl2_ref, bl2_ref, wr2_ref)


def kernel(x, adj, W_l0, b_l0, W_r0, W_l1, b_l1, W_r1, W_l2, b_l2, W_r2):
    n, _ = x.shape
    d_out = W_l2.shape[1]
    g0 = jnp.concatenate([x.T, jnp.ones((1, n), x.dtype)], axis=0)
    outT = pl.pallas_call(
        _sage_fused,
        out_shape=jax.ShapeDtypeStruct((d_out, n), jnp.float32),
    )(adj.astype(jnp.bfloat16), g0.astype(jnp.bfloat16),
      W_l0.T, b_l0.reshape(-1, 1), W_r0.T,
      W_l1.T, b_l1.reshape(-1, 1), W_r1.T,
      W_l2.T, b_l2.reshape(-1, 1), W_r2.T)
    return outT.T


# speedup vs baseline: 1.2206x; 1.2206x over previous
import jax
import jax.numpy as jnp
from jax.experimental import pallas as pl


def _sage_fused(adj_ref, gx_ref,
                wl0_ref, bl0_ref, wr0_ref,
                wl1_ref, bl1_ref, wr1_ref,
                wl2_ref, bl2_ref, wr2_ref,
                out_ref):
    adj = adj_ref[...]
    g0 = gx_ref[...]
    d = g0.shape[0] - 1

    first = jnp.dot(g0, adj, preferred_element_type=jnp.float32)
    dinv = 1.0 / jnp.maximum(first[d:, :], 1.0)

    def tail(gh, aggT, wlT_ref, blc_ref, wrT_ref):
        meanT = aggT * dinv
        lin_l = jnp.dot(wlT_ref[...], meanT, preferred_element_type=jnp.float32)
        lin_r = jnp.dot(wrT_ref[...], gh, preferred_element_type=jnp.float32)
        return lin_l + lin_r + blc_ref[...]

    g = jnp.maximum(tail(g0[:d, :], first[:d, :],
                         wl0_ref, bl0_ref, wr0_ref), 0.0)
    aggT = jnp.dot(g, adj, preferred_element_type=jnp.float32)
    g = jnp.maximum(tail(g, aggT, wl1_ref, bl1_ref, wr1_ref), 0.0)
    aggT = jnp.dot(g, adj, preferred_element_type=jnp.float32)
    out_ref[...] = tail(g, aggT, wl2_ref, bl2_ref, wr2_ref)


def kernel(x, adj, W_l0, b_l0, W_r0, W_l1, b_l1, W_r1, W_l2, b_l2, W_r2):
    n, _ = x.shape
    d_out = W_l2.shape[1]
    g0 = jnp.concatenate([x.T, jnp.ones((1, n), x.dtype)], axis=0)
    outT = pl.pallas_call(
        _sage_fused,
        out_shape=jax.ShapeDtypeStruct((d_out, n), jnp.float32),
    )(adj, g0,
      W_l0.T, b_l0.reshape(-1, 1), W_r0.T,
      W_l1.T, b_l1.reshape(-1, 1), W_r1.T,
      W_l2.T, b_l2.reshape(-1, 1), W_r2.T)
    return outT.T


# speedup vs baseline: 2.1647x; 1.7735x over previous
import jax
import jax.numpy as jnp
from jax.experimental import pallas as pl


def _sage_fused(adj_ref, x_ref,
                wl0_ref, bl0_ref, wr0_ref,
                wl1_ref, bl1_ref, wr1_ref,
                wl2_ref, bl2_ref, wr2_ref,
                out_ref):
    adj = adj_ref[...]
    g = jnp.transpose(x_ref[...])

    deg = jnp.sum(adj, axis=0, keepdims=True)
    dinv = 1.0 / jnp.maximum(deg, 1.0)

    def layer(gh, wl_ref, bl_ref, wr_ref):
        aggT = jnp.dot(gh, adj, preferred_element_type=jnp.float32)
        meanT = aggT * dinv
        lin_l = jnp.dot(jnp.transpose(wl_ref[...]), meanT,
                        preferred_element_type=jnp.float32)
        lin_r = jnp.dot(jnp.transpose(wr_ref[...]), gh,
                        preferred_element_type=jnp.float32)
        return lin_l + lin_r + jnp.transpose(bl_ref[...])

    g = jnp.maximum(layer(g, wl0_ref, bl0_ref, wr0_ref), 0.0)
    g = jnp.maximum(layer(g, wl1_ref, bl1_ref, wr1_ref), 0.0)
    out_ref[...] = jnp.transpose(layer(g, wl2_ref, bl2_ref, wr2_ref))


def kernel(x, adj, W_l0, b_l0, W_r0, W_l1, b_l1, W_r1, W_l2, b_l2, W_r2):
    n, _ = x.shape
    d_out = W_l2.shape[1]
    return pl.pallas_call(
        _sage_fused,
        out_shape=jax.ShapeDtypeStruct((n, d_out), jnp.float32),
    )(adj, x,
      W_l0, b_l0.reshape(1, -1), W_r0,
      W_l1, b_l1.reshape(1, -1), W_r1,
      W_l2, b_l2.reshape(1, -1), W_r2)
